# Initial kernel scaffold; baseline (speedup 1.0000x reference)
import jax
import jax.numpy as jnp
from jax.experimental import pallas as pl


def kernel(src_treats, dst_treats, src_treated_by, dst_treated_by, embed_chemical, embed_disease, W1_treats, b1_treats, W1_treated_by, b1_treated_by, W2_treats, b2_treats, W2_treated_by, b2_treated_by):
    raise NotImplementedError("write your pallas kernel here")


# speedup vs baseline: 3.8337x; 3.8337x over previous
import functools

import jax
import jax.numpy as jnp
from jax import lax
from jax.experimental import pallas as pl
from jax.experimental.pallas import tpu as pltpu
from jax.experimental.pallas import tpu_sc as plsc

N_NODE = 10000
E = 320000
D = 128
D_TAB = D + 16
R_PAD = 10240
NW = 32
CH = 128
CHUNKS_PER_W = 79
E_PAD = NW * CHUNKS_PER_W * CH
PAD_DST = N_NODE + 64


def _sc_edge_pass():
    mesh = plsc.VectorSubcoreMesh(core_axis_name="c", subcore_axis_name="s")

    @functools.partial(
        pl.kernel,
        out_type=jax.ShapeDtypeStruct((2, R_PAD, D_TAB), jnp.float32),
        mesh=mesh,
        scratch_types=[
            pltpu.VMEM((CH,), jnp.int32),
            pltpu.VMEM((CH,), jnp.int32),
            pltpu.VMEM((CH, D_TAB), jnp.float32),
            pltpu.VMEM_SHARED((R_PAD, D_TAB), jnp.float32),
            pltpu.SemaphoreType.DMA,
        ],
        compiler_params=pltpu.CompilerParams(use_tc_tiling_on_sc=False),
    )
    def k(table_hbm, src_hbm, dst_hbm, zeros_hbm, out_hbm,
          sidx, didx, rows, acc, sem):
        c = lax.axis_index("c")
        s = lax.axis_index("s")
        wid = c * 16 + s

        pltpu.sync_copy(zeros_hbm, acc.at[pl.ds(s * 640, 640)])
        plsc.subcore_barrier()

        def body(g, _):
            off = wid * (CHUNKS_PER_W * CH) + g * CH
            pltpu.sync_copy(src_hbm.at[pl.ds(off, CH)], sidx)
            pltpu.sync_copy(dst_hbm.at[pl.ds(off, CH)], didx)
            pltpu.async_copy(table_hbm.at[sidx], rows, sem).wait()
            pltpu.sync_copy(rows, acc.at[didx], add=True)
            return 0

        lax.fori_loop(0, CHUNKS_PER_W, body, 0)
        plsc.subcore_barrier()
        pltpu.sync_copy(acc.at[pl.ds(s * 640, 640)],
                        out_hbm.at[c, pl.ds(s * 640, 640)])

    return k


_BLK = 1280


def _tc_project_kernel(x_ref, w_ref, b_ref, o_ref):
    wh = jnp.dot(x_ref[...], w_ref[...], preferred_element_type=jnp.float32)
    o_ref[:, :D] = wh + b_ref[...]
    o_ref[:, D:] = jnp.ones((_BLK, D_TAB - D), jnp.float32)


def _tc_mean_project_kernel(a_ref, w_ref, b_ref, o_ref):
    p = a_ref[0] + a_ref[1]
    cnt = jnp.maximum(p[:, D:D + 1], 1.0)
    h = p[:, :D] / cnt
    h = jnp.where(h >= 0, h, 0.01 * h)
    wh = jnp.dot(h, w_ref[...], preferred_element_type=jnp.float32)
    o_ref[:, :D] = wh + b_ref[...]
    o_ref[:, D:] = jnp.ones((_BLK, D_TAB - D), jnp.float32)


def _tc_mean_kernel(a_ref, o_ref):
    p = a_ref[0] + a_ref[1]
    cnt = jnp.maximum(p[:, D:D + 1], 1.0)
    o_ref[...] = p[:, :D] / cnt


def _tc_project(x, w, b):
    return pl.pallas_call(
        _tc_project_kernel,
        grid=(R_PAD // _BLK,),
        in_specs=[
            pl.BlockSpec((_BLK, D), lambda i: (i, 0)),
            pl.BlockSpec((D, D), lambda i: (0, 0)),
            pl.BlockSpec((1, D), lambda i: (0, 0)),
        ],
        out_specs=pl.BlockSpec((_BLK, D_TAB), lambda i: (i, 0)),
        out_shape=jax.ShapeDtypeStruct((R_PAD, D_TAB), jnp.float32),
    )(x, w, b)


def _tc_mean_project(acc, w, b):
    return pl.pallas_call(
        _tc_mean_project_kernel,
        grid=(R_PAD // _BLK,),
        in_specs=[
            pl.BlockSpec((2, _BLK, D_TAB), lambda i: (0, i, 0)),
            pl.BlockSpec((D, D), lambda i: (0, 0)),
            pl.BlockSpec((1, D), lambda i: (0, 0)),
        ],
        out_specs=pl.BlockSpec((_BLK, D_TAB), lambda i: (i, 0)),
        out_shape=jax.ShapeDtypeStruct((R_PAD, D_TAB), jnp.float32),
    )(acc, w, b)


def _tc_mean(acc):
    return pl.pallas_call(
        _tc_mean_kernel,
        grid=(R_PAD // _BLK,),
        in_specs=[pl.BlockSpec((2, _BLK, D_TAB), lambda i: (0, i, 0))],
        out_specs=pl.BlockSpec((_BLK, D), lambda i: (i, 0)),
        out_shape=jax.ShapeDtypeStruct((R_PAD, D), jnp.float32),
    )(acc)


def kernel(src_treats, dst_treats, src_treated_by, dst_treated_by,
           embed_chemical, embed_disease,
           W1_treats, b1_treats, W1_treated_by, b1_treated_by,
           W2_treats, b2_treats, W2_treated_by, b2_treated_by):
    del embed_disease, W1_treated_by, b1_treated_by, W2_treats, b2_treats

    pad_e = E_PAD - E
    src1 = jnp.pad(src_treats, (0, pad_e))
    dst1 = jnp.pad(dst_treats, (0, pad_e), constant_values=PAD_DST)
    src2 = jnp.pad(src_treated_by, (0, pad_e))
    dst2 = jnp.pad(dst_treated_by, (0, pad_e), constant_values=PAD_DST)

    x = jnp.pad(embed_chemical, ((0, R_PAD - N_NODE), (0, 0)))
    zeros = jnp.zeros((640, D_TAB), jnp.float32)

    edge_pass = _sc_edge_pass()

    table1 = _tc_project(x, W1_treats, b1_treats.reshape(1, D))
    acc1 = edge_pass(table1, src1, dst1, zeros)
    table2 = _tc_mean_project(acc1, W2_treated_by, b2_treated_by.reshape(1, D))
    acc2 = edge_pass(table2, src2, dst2, zeros)
    out = _tc_mean(acc2)
    return out[:N_NODE]
